# Initial kernel scaffold; baseline (speedup 1.0000x reference)
import jax
import jax.numpy as jnp
from jax.experimental import pallas as pl


def kernel(x, edge_index, edge_attr, batch, Wl1, bl1, Wr1, br1, We1, att1, b1, Wl2, bl2, Wr2, br2, We2, att2, b2, Wl3, bl3, Wr3, br3, We3, att3, b3, Wlin, blin):
    raise NotImplementedError("write your pallas kernel here")


# speedup vs baseline: 3.6905x; 3.6905x over previous
import functools

import jax
import jax.numpy as jnp
from jax import lax
from jax.experimental import pallas as pl
from jax.experimental.pallas import tpu as pltpu
from jax.experimental.pallas import tpu_sc as plsc

N = 10000
E = 160000
G = 64

NPAD = 10240
EPAD = 163840
NW = 32
EPT = EPAD // NW
GSZ = 128
NGRP = EPT // GSZ
NPT = NPAD // 16

@functools.lru_cache(maxsize=None)
def _get_mesh():
    return plsc.VectorSubcoreMesh(
        core_axis_name="c", subcore_axis_name="s",
        num_cores=2, num_subcores=16)


def _wid():
    return lax.axis_index("c") * 16 + lax.axis_index("s")


def _edge_mm_body(ea_ref, w_ref, e1_ref, e2_ref, e3_ref):
    e = jnp.dot(ea_ref[...], w_ref[...], preferred_element_type=jnp.float32)
    e1_ref[...] = e[:, 0:128]
    e2_ref[...] = e[:, 128:160]
    e3_ref[...] = e[:, 160:176]


def _node_mm_body(x_ref, w_ref, b_ref, xl_ref, xr_ref):
    dout = xl_ref.shape[1]
    y = jnp.dot(x_ref[...], w_ref[...], preferred_element_type=jnp.float32)
    y = y + b_ref[...]
    xl_ref[...] = y[:, 0:dout]
    xr_ref[...] = y[:, dout:2 * dout]


def _attn_body(xls_ref, xrd_ref, e_ref, att_ref, a_ref):
    m = xls_ref[...] + xrd_ref[...] + e_ref[...]
    m = jnp.where(m >= 0.0, m, 0.2 * m)
    a = jnp.sum(m * att_ref[...], axis=1)
    a_ref[...] = a.reshape(a_ref.shape)


def _softnum_body(a_ref, ae_ref):
    a = a_ref[...]
    amax = jnp.max(a)
    nrow = a.shape[0]
    ae_ref[0:nrow, :] = jnp.exp(a - amax)
    ae_ref[nrow:, :] = jnp.zeros((ae_ref.shape[0] - nrow, a.shape[1]),
                                 jnp.float32)


def _invden_body(d_ref, inv_ref):
    h = d_ref.shape[0] // 2
    den = d_ref[0:h, :] + d_ref[h:, :]
    inv_ref[...] = 1.0 / (den + 1e-16)


def _vscale_body(alpha_ref, xls_ref, v_ref):
    v_ref[...] = alpha_ref[...] * xls_ref[...]


def _combine_body(p0_ref, p1_ref, b_ref, h_ref):
    y = p0_ref[0] + p1_ref[0] + b_ref[...]
    h_ref[...] = jnp.maximum(y, 0.0)


def _pool_body(h_ref, batch_ref, wlin_ref, blin_ref, o_ref):
    b = batch_ref[...]
    gids = lax.broadcasted_iota(jnp.int32, (G, b.shape[1]), 0)
    onehot = (b == gids).astype(jnp.float32)
    sums = jnp.dot(onehot, h_ref[...], preferred_element_type=jnp.float32)
    cnt = jnp.sum(onehot, axis=1)
    pooled = sums / jnp.clip(cnt, 1.0)[:, None]
    z = jnp.dot(pooled, wlin_ref[...], preferred_element_type=jnp.float32)
    o_ref[...] = jax.nn.sigmoid(z + blin_ref[...])


def _edge_mm(ea, wall):
    nblk = E // 640
    return pl.pallas_call(
        _edge_mm_body,
        grid=(nblk,),
        in_specs=[pl.BlockSpec((640, 512), lambda i: (i, 0)),
                  pl.BlockSpec((512, 176), lambda i: (0, 0))],
        out_specs=[pl.BlockSpec((640, 128), lambda i: (i, 0)),
                   pl.BlockSpec((640, 32), lambda i: (i, 0)),
                   pl.BlockSpec((640, 16), lambda i: (i, 0))],
        out_shape=[jax.ShapeDtypeStruct((E, 128), jnp.float32),
                   jax.ShapeDtypeStruct((E, 32), jnp.float32),
                   jax.ShapeDtypeStruct((E, 16), jnp.float32)],
    )(ea, wall)


def _node_mm(h, w, b, din, dout):
    nblk = NPAD // 512
    return pl.pallas_call(
        _node_mm_body,
        grid=(nblk,),
        in_specs=[pl.BlockSpec((512, din), lambda i: (i, 0)),
                  pl.BlockSpec((din, 2 * dout), lambda i: (0, 0)),
                  pl.BlockSpec((1, 2 * dout), lambda i: (0, 0))],
        out_specs=[pl.BlockSpec((512, dout), lambda i: (i, 0)),
                   pl.BlockSpec((512, dout), lambda i: (i, 0))],
        out_shape=[jax.ShapeDtypeStruct((NPAD, dout), jnp.float32),
                   jax.ShapeDtypeStruct((NPAD, dout), jnp.float32)],
    )(h, w, b)


def _attn(xls, xrd, e, att, dout):
    nblk = E // 640
    a = pl.pallas_call(
        _attn_body,
        grid=(nblk,),
        in_specs=[pl.BlockSpec((640, dout), lambda i: (i, 0)),
                  pl.BlockSpec((640, dout), lambda i: (i, 0)),
                  pl.BlockSpec((640, dout), lambda i: (i, 0)),
                  pl.BlockSpec((1, dout), lambda i: (0, 0))],
        out_specs=pl.BlockSpec((1, 5, 128), lambda i: (i, 0, 0)),
        out_shape=jax.ShapeDtypeStruct((nblk, 5, 128), jnp.float32),
    )(xls, xrd, e, att)
    return a.reshape(E // 128, 128)


def _softnum(a):
    ae = pl.pallas_call(
        _softnum_body,
        in_specs=[pl.BlockSpec((E // 128, 128), lambda: (0, 0))],
        out_specs=pl.BlockSpec((EPAD // 128, 128), lambda: (0, 0)),
        out_shape=jax.ShapeDtypeStruct((EPAD // 128, 128), jnp.float32),
    )(a)
    return ae.reshape(EPAD)


def _invden(denp):
    d2 = denp.reshape(2 * NPAD // 128, 128)
    inv = pl.pallas_call(
        _invden_body,
        in_specs=[pl.BlockSpec((2 * NPAD // 128, 128), lambda: (0, 0))],
        out_specs=pl.BlockSpec((NPAD // 128, 128), lambda: (0, 0)),
        out_shape=jax.ShapeDtypeStruct((NPAD // 128, 128), jnp.float32),
    )(d2)
    return inv.reshape(NPAD)


def _vscale(alpha, xls, dout):
    nblk = EPAD // 1024
    return pl.pallas_call(
        _vscale_body,
        grid=(nblk,),
        in_specs=[pl.BlockSpec((1024, 1), lambda i: (i, 0)),
                  pl.BlockSpec((1024, dout), lambda i: (i, 0))],
        out_specs=pl.BlockSpec((1024, dout), lambda i: (i, 0)),
        out_shape=jax.ShapeDtypeStruct((EPAD, dout), jnp.float32),
    )(alpha.reshape(EPAD, 1), xls)


def _combine(outp, b, dout):
    nblk = NPAD // 1024
    return pl.pallas_call(
        _combine_body,
        grid=(nblk,),
        in_specs=[pl.BlockSpec((1, 1024, dout), lambda i: (0, i, 0)),
                  pl.BlockSpec((1, 1024, dout), lambda i: (1, i, 0)),
                  pl.BlockSpec((1, dout), lambda i: (0, 0))],
        out_specs=pl.BlockSpec((1024, dout), lambda i: (i, 0)),
        out_shape=jax.ShapeDtypeStruct((NPAD, dout), jnp.float32),
    )(outp, outp, b)


def _pool(h, batch2d, wlin, blin):
    return pl.pallas_call(
        _pool_body,
        in_specs=[pl.BlockSpec((NPAD, 16), lambda: (0, 0)),
                  pl.BlockSpec((1, NPAD), lambda: (0, 0)),
                  pl.BlockSpec((16, 1), lambda: (0, 0)),
                  pl.BlockSpec((1, 1), lambda: (0, 0))],
        out_specs=pl.BlockSpec((G, 1), lambda: (0, 0)),
        out_shape=jax.ShapeDtypeStruct((G, 1), jnp.float32),
    )(h, batch2d, wlin, blin)


@functools.lru_cache(maxsize=None)
def _sc_gather(dout):
    @functools.partial(
        pl.kernel,
        out_type=[jax.ShapeDtypeStruct((EPAD, dout), jnp.float32),
                  jax.ShapeDtypeStruct((EPAD, dout), jnp.float32)],
        mesh=_get_mesh(),
        scratch_types=[pltpu.VMEM((GSZ,), jnp.int32),
                       pltpu.VMEM((GSZ, dout), jnp.float32),
                       pltpu.VMEM((GSZ,), jnp.int32),
                       pltpu.VMEM((GSZ, dout), jnp.float32),
                       pltpu.SemaphoreType.DMA,
                       pltpu.SemaphoreType.DMA],
        compiler_params=pltpu.CompilerParams(use_tc_tiling_on_sc=False),
    )
    def k(xl_hbm, xr_hbm, src_hbm, dst_hbm, oxl, oxr,
          idx_s, rows_s, idx_d, rows_d, sem_s, sem_d):
        base = _wid() * EPT

        def body(g, carry):
            off = base + g * GSZ
            pltpu.sync_copy(src_hbm.at[pl.ds(off, GSZ)], idx_s)
            pltpu.sync_copy(dst_hbm.at[pl.ds(off, GSZ)], idx_d)
            cs = pltpu.async_copy(xl_hbm.at[idx_s], rows_s, sem_s)
            cd = pltpu.async_copy(xr_hbm.at[idx_d], rows_d, sem_d)
            cs.wait()
            cd.wait()
            pltpu.sync_copy(rows_s, oxl.at[pl.ds(off, GSZ)])
            pltpu.sync_copy(rows_d, oxr.at[pl.ds(off, GSZ)])
            return carry

        lax.fori_loop(0, NGRP, body, 0)

    return k


@functools.lru_cache(maxsize=None)
def _sc_den_k():
    @functools.partial(
        pl.kernel,
        out_type=jax.ShapeDtypeStruct((2 * NPAD,), jnp.float32),
        mesh=_get_mesh(),
        scratch_types=[pltpu.VMEM_SHARED((NPAD,), jnp.float32),
                       pltpu.VMEM((GSZ,), jnp.float32),
                       pltpu.VMEM((GSZ,), jnp.int32)],
    )
    def k(ae_hbm, dst_hbm, zeros_hbm, out, shared_den, ae_g, idx_g):
        cid = lax.axis_index("c")
        sid = lax.axis_index("s")
        base = _wid() * EPT
        nslice = pl.ds(sid * NPT, NPT)
        pltpu.sync_copy(zeros_hbm.at[nslice], shared_den.at[nslice])
        plsc.subcore_barrier()

        def body(g, carry):
            off = base + g * GSZ
            pltpu.sync_copy(ae_hbm.at[pl.ds(off, GSZ)], ae_g)
            pltpu.sync_copy(dst_hbm.at[pl.ds(off, GSZ)], idx_g)
            pltpu.sync_copy(ae_g, shared_den.at[idx_g], add=True)
            return carry

        lax.fori_loop(0, NGRP, body, 0)
        plsc.subcore_barrier()
        pltpu.sync_copy(shared_den.at[nslice],
                        out.at[pl.ds(cid * NPAD + sid * NPT, NPT)])

    return k


@functools.lru_cache(maxsize=None)
def _sc_alpha_k():
    @functools.partial(
        pl.kernel,
        out_type=jax.ShapeDtypeStruct((EPAD,), jnp.float32),
        mesh=_get_mesh(),
        scratch_types=[pltpu.VMEM((GSZ,), jnp.int32),
                       pltpu.VMEM((GSZ,), jnp.float32),
                       pltpu.VMEM((GSZ,), jnp.float32),
                       pltpu.VMEM((GSZ,), jnp.float32),
                       pltpu.SemaphoreType.DMA],
    )
    def k(ae_hbm, invd_hbm, dst_hbm, out, idx_g, vals_g, ae_g, alpha_g, sem):
        base = _wid() * EPT

        def body(g, carry):
            off = base + g * GSZ
            pltpu.sync_copy(dst_hbm.at[pl.ds(off, GSZ)], idx_g)
            pltpu.sync_copy(ae_hbm.at[pl.ds(off, GSZ)], ae_g)
            pltpu.async_copy(invd_hbm.at[idx_g], vals_g, sem).wait()
            for j in range(GSZ // 16):
                s = pl.ds(j * 16, 16)
                alpha_g[s] = ae_g[s] * vals_g[s]
            pltpu.sync_copy(alpha_g, out.at[pl.ds(off, GSZ)])
            return carry

        lax.fori_loop(0, NGRP, body, 0)

    return k


@functools.lru_cache(maxsize=None)
def _sc_row_scatter(dout):
    @functools.partial(
        pl.kernel,
        out_type=jax.ShapeDtypeStruct((2 * NPAD, dout), jnp.float32),
        mesh=_get_mesh(),
        scratch_types=[pltpu.VMEM_SHARED((NPAD, dout), jnp.float32),
                       pltpu.VMEM((GSZ, dout), jnp.float32),
                       pltpu.VMEM((GSZ,), jnp.int32)],
    )
    def k(v_hbm, dst_hbm, zeros_hbm, out, shared_acc, rows_v, idx_g):
        cid = lax.axis_index("c")
        sid = lax.axis_index("s")
        base = _wid() * EPT
        nslice = pl.ds(sid * NPT, NPT)
        pltpu.sync_copy(zeros_hbm.at[nslice], shared_acc.at[nslice])
        plsc.subcore_barrier()

        def body(g, carry):
            off = base + g * GSZ
            pltpu.sync_copy(v_hbm.at[pl.ds(off, GSZ)], rows_v)
            pltpu.sync_copy(dst_hbm.at[pl.ds(off, GSZ)], idx_g)
            pltpu.sync_copy(rows_v, shared_acc.at[idx_g], add=True)
            return carry

        lax.fori_loop(0, NGRP, body, 0)
        plsc.subcore_barrier()
        pltpu.sync_copy(shared_acc.at[nslice],
                        out.at[pl.ds(cid * NPAD + sid * NPT, NPT)])

    return k


def _gat_layer(h, e, srcp, dstp, w, b, att2d, bias, din, dout):
    xl, xr = _node_mm(h, w, b, din, dout)
    xls, xrd = _sc_gather(dout)(xl, xr, srcp, dstp)
    a = _attn(xls, xrd, e, att2d, dout)
    ae = _softnum(a)
    zeros1 = jnp.zeros((NPAD,), jnp.float32)
    denp = _sc_den_k()(ae, dstp, zeros1)
    invd = _invden(denp)
    alpha = _sc_alpha_k()(ae, invd, dstp)
    v = _vscale(alpha, xls, dout)
    zerosd = jnp.zeros((NPAD, dout), jnp.float32)
    outp = _sc_row_scatter(dout)(v, dstp, zerosd)
    hn = _combine(outp.reshape(2, NPAD, dout), bias, dout)
    return hn


def kernel(x, edge_index, edge_attr, batch,
           Wl1, bl1, Wr1, br1, We1, att1, b1,
           Wl2, bl2, Wr2, br2, We2, att2, b2,
           Wl3, bl3, Wr3, br3, We3, att3, b3,
           Wlin, blin):
    f32 = jnp.float32
    src = edge_index[0].astype(jnp.int32)
    dst = edge_index[1].astype(jnp.int32)
    padi = (jnp.arange(EPAD - E, dtype=jnp.int32) * 37) % N
    srcp = jnp.concatenate([src, padi])
    dstp = jnp.concatenate([dst, padi])

    xp = jnp.zeros((NPAD, 512), f32).at[:N].set(x)
    batchp = jnp.full((1, NPAD), G, jnp.int32).at[0, :N].set(
        batch.astype(jnp.int32))

    Wl3p = jnp.zeros((32, 16), f32).at[:, :8].set(Wl3)
    Wr3p = jnp.zeros((32, 16), f32).at[:, :8].set(Wr3)
    We3p = jnp.zeros((512, 16), f32).at[:, :8].set(We3)
    att3p = jnp.zeros((16,), f32).at[:8].set(att3)
    b3p = jnp.zeros((16,), f32).at[:8].set(b3)
    bl3p = jnp.zeros((16,), f32).at[:8].set(bl3)
    br3p = jnp.zeros((16,), f32).at[:8].set(br3)
    Wlinp = jnp.zeros((16, 1), f32).at[:8].set(Wlin)

    Wall = jnp.concatenate([We1, We2, We3p], axis=1)
    e1, e2, e3 = _edge_mm(edge_attr, Wall)

    w1 = jnp.concatenate([Wl1, Wr1], axis=1)
    bb1 = jnp.concatenate([bl1, br1]).reshape(1, 256)
    w2 = jnp.concatenate([Wl2, Wr2], axis=1)
    bb2 = jnp.concatenate([bl2, br2]).reshape(1, 64)
    w3 = jnp.concatenate([Wl3p, Wr3p], axis=1)
    bb3 = jnp.concatenate([bl3p, br3p]).reshape(1, 32)

    h = _gat_layer(xp, e1, srcp, dstp, w1, bb1, att1.reshape(1, 128),
                   b1.reshape(1, 128), 512, 128)
    h = _gat_layer(h, e2, srcp, dstp, w2, bb2, att2.reshape(1, 32),
                   b2.reshape(1, 32), 128, 32)
    h = _gat_layer(h, e3, srcp, dstp, w3, bb3, att3p.reshape(1, 16),
                   b3p.reshape(1, 16), 32, 16)

    return _pool(h, batchp, Wlinp, blin.reshape(1, 1))
